# baseline (speedup 1.0000x reference)
import functools

import jax
import jax.numpy as jnp
from jax import lax
from jax.experimental import pallas as pl
from jax.experimental.pallas import tpu as pltpu

_CUTOFF = 5.0


def _swish(x):
    return x * jax.nn.sigmoid(x)


def _wf_body(fij_ref, rij_ref, W1_ref, b1_ref, W2_ref, b2_ref, out_ref):
    h = jnp.dot(fij_ref[...], W1_ref[...], preferred_element_type=jnp.float32)
    h = _swish(h + b1_ref[...])
    wf = jnp.dot(h, W2_ref[...], preferred_element_type=jnp.float32) + b2_ref[...]
    r = rij_ref[...]
    c = jnp.where(r < _CUTOFF, 0.5 * (jnp.cos(r * (jnp.pi / _CUTOFF)) + 1.0), 0.0)
    out_ref[...] = wf * c


def _compute_wf(fij, rij2, W1, b1r, W2, b2r, blk):
    e_pad = fij.shape[0]
    grid = e_pad // blk
    return pl.pallas_call(
        _wf_body,
        grid=(grid,),
        in_specs=[
            pl.BlockSpec((blk, fij.shape[1]), lambda i: (i, 0)),
            pl.BlockSpec((blk, 1), lambda i: (i, 0)),
            pl.BlockSpec(W1.shape, lambda i: (0, 0)),
            pl.BlockSpec(b1r.shape, lambda i: (0, 0)),
            pl.BlockSpec(W2.shape, lambda i: (0, 0)),
            pl.BlockSpec(b2r.shape, lambda i: (0, 0)),
        ],
        out_specs=pl.BlockSpec((blk, 64), lambda i: (i, 0)),
        out_shape=jax.ShapeDtypeStruct((e_pad, 64), jnp.float32),
    )(fij, rij2, W1, b1r, W2, b2r)


def _proj_body(x_ref, W_ref, out_ref):
    out_ref[...] = jnp.dot(x_ref[...], W_ref[...],
                           preferred_element_type=jnp.float32)


def _compute_y(feat, W_in2f, blk):
    n = feat.shape[0]
    return pl.pallas_call(
        _proj_body,
        grid=(n // blk,),
        in_specs=[
            pl.BlockSpec((blk, feat.shape[1]), lambda i: (i, 0)),
            pl.BlockSpec(W_in2f.shape, lambda i: (0, 0)),
        ],
        out_specs=pl.BlockSpec((blk, 64), lambda i: (i, 0)),
        out_shape=jax.ShapeDtypeStruct((n, 64), jnp.float32),
    )(feat, W_in2f)


def _out_body(a0_ref, a1_ref, Wa_ref, Wb_ref, b_ref, out_ref):
    acc = jnp.dot(a0_ref[...], Wa_ref[...], preferred_element_type=jnp.float32)
    acc += jnp.dot(a1_ref[...], Wb_ref[...], preferred_element_type=jnp.float32)
    out_ref[...] = _swish(acc + b_ref[...])


def _compute_out(agg0, agg1, Wa, Wb, br, blk):
    n = agg0.shape[0]
    return pl.pallas_call(
        _out_body,
        grid=(n // blk,),
        in_specs=[
            pl.BlockSpec((blk, 32), lambda i: (i, 0)),
            pl.BlockSpec((blk, 32), lambda i: (i, 0)),
            pl.BlockSpec(Wa.shape, lambda i: (0, 0)),
            pl.BlockSpec(Wb.shape, lambda i: (0, 0)),
            pl.BlockSpec(br.shape, lambda i: (0, 0)),
        ],
        out_specs=pl.BlockSpec((blk, 64), lambda i: (i, 0)),
        out_shape=jax.ShapeDtypeStruct((n, 64), jnp.float32),
    )(agg0, agg1, Wa, Wb, br)


def kernel(feat, edge_index, fij, rij, W1, b1, W2, b2, W_in2f, W_f2out, b_f2out):
    n = feat.shape[0]
    e = fij.shape[0]
    e_pad = ((e + 128 * 128 - 1) // (128 * 128)) * (128 * 128)
    pad = e_pad - e

    src = edge_index[0]
    dst = edge_index[1]
    fij_p = jnp.pad(fij, ((0, pad), (0, 0)))
    rij_p = jnp.pad(rij, (0, pad), constant_values=2.0 * _CUTOFF)
    src_p = jnp.pad(src, (0, pad))
    dst_p = jnp.pad(dst, (0, pad))

    wf = _compute_wf(fij_p, rij_p[:, None], W1, b1[None, :], W2, b2[None, :],
                     blk=2048)
    y = _compute_y(feat, W_in2f, blk=1000)

    m = jnp.take(y, src_p, axis=0) * wf
    agg = jax.ops.segment_sum(m, dst_p, num_segments=n)

    out = _compute_out(agg[:, :32], agg[:, 32:], W_f2out[:32], W_f2out[32:],
                       b_f2out[None, :], blk=1000)
    return out


# speedup vs baseline: 1.7608x; 1.7608x over previous
import functools

import jax
import jax.numpy as jnp
from jax import lax
from jax.experimental import pallas as pl
from jax.experimental.pallas import tpu as pltpu
import jax.experimental.pallas.tpu_sc as plsc

_CUTOFF = 5.0


def _swish(x):
    return x * jax.nn.sigmoid(x)


def _wf_body(fij_ref, rij_ref, W1_ref, b1_ref, W2_ref, b2_ref, *out_refs):
    h = jnp.dot(fij_ref[...], W1_ref[...], preferred_element_type=jnp.float32)
    h = _swish(h + b1_ref[...])
    wf = jnp.dot(h, W2_ref[...], preferred_element_type=jnp.float32) + b2_ref[...]
    r = rij_ref[...]
    c = jnp.where(r < _CUTOFF, 0.5 * (jnp.cos(r * (jnp.pi / _CUTOFF)) + 1.0), 0.0)
    wf = wf * c
    for q in range(4):
        out_refs[q][...] = wf[:, 16 * q:16 * (q + 1)]


def _compute_wf(fij, rij2, W1, b1r, W2, b2r, blk):
    e_pad = fij.shape[0]
    quarter = jax.ShapeDtypeStruct((e_pad, 16), jnp.float32)
    return pl.pallas_call(
        _wf_body,
        grid=(e_pad // blk,),
        in_specs=[
            pl.BlockSpec((blk, fij.shape[1]), lambda i: (i, 0)),
            pl.BlockSpec((blk, 1), lambda i: (i, 0)),
            pl.BlockSpec(W1.shape, lambda i: (0, 0)),
            pl.BlockSpec(b1r.shape, lambda i: (0, 0)),
            pl.BlockSpec(W2.shape, lambda i: (0, 0)),
            pl.BlockSpec(b2r.shape, lambda i: (0, 0)),
        ],
        out_specs=[pl.BlockSpec((blk, 16), lambda i: (i, 0))] * 4,
        out_shape=[quarter] * 4,
    )(fij, rij2, W1, b1r, W2, b2r)


def _proj_body(x_ref, W_ref, *out_refs):
    y = jnp.dot(x_ref[...], W_ref[...], preferred_element_type=jnp.float32)
    for q in range(4):
        out_refs[q][...] = y[:, 16 * q:16 * (q + 1)]


def _compute_y(feat, W_in2f, blk):
    n = feat.shape[0]
    quarter = jax.ShapeDtypeStruct((n, 16), jnp.float32)
    return pl.pallas_call(
        _proj_body,
        grid=(n // blk,),
        in_specs=[
            pl.BlockSpec((blk, feat.shape[1]), lambda i: (i, 0)),
            pl.BlockSpec(W_in2f.shape, lambda i: (0, 0)),
        ],
        out_specs=[pl.BlockSpec((blk, 16), lambda i: (i, 0))] * 4,
        out_shape=[quarter] * 4,
    )(feat, W_in2f)


def _sc_middle(src_mat, dst_mat, yq, wfq):
    n = yq[0].shape[0]
    n_chunks = src_mat.shape[0]
    sb = 4
    chunks_per_tile = n_chunks // 16
    n_batches = chunks_per_tile // sb
    oc = 400
    n_oc = n // oc
    assert chunks_per_tile % sb == 0 and n % oc == 0 and oc % 8 == 0
    assert oc <= sb * 128

    mesh = plsc.VectorSubcoreMesh(core_axis_name="c", subcore_axis_name="s")
    quarter = jax.ShapeDtypeStruct((n, 16), jnp.float32)

    @functools.partial(
        pl.kernel,
        out_type=[quarter] * 4,
        mesh=mesh,
        compiler_params=pltpu.CompilerParams(use_tc_tiling_on_sc=False),
        scratch_types=[
            pltpu.VMEM((sb, 128), jnp.int32),
            pltpu.VMEM((sb, 128), jnp.int32),
            pltpu.VMEM((sb * 128, 16), jnp.float32),
            pltpu.VMEM((sb * 128, 16), jnp.float32),
            pltpu.VMEM_SHARED((n, 16), jnp.float32),
            pltpu.VMEM_SHARED((n, 16), jnp.float32),
            pltpu.SemaphoreType.DMA,
        ],
    )
    def body(src_hbm, dst_hbm, y0, y1, y2, y3, w0, w1, w2, w3,
             o0, o1, o2, o3, src_v, dst_v, wf_v, rows_v, y_sp, agg_sp, sem):
        c = lax.axis_index("c")
        s = lax.axis_index("s")
        ys = ((y0, y2), (y1, y3))
        ws = ((w0, w2), (w1, w3))
        outs = ((o0, o2), (o1, o3))
        zeros16 = jnp.zeros((16,), jnp.float32)
        nk = (n_oc - s + 15) // 16

        for p in range(2):
            def zr(i, carry):
                rows_v[i, pl.ds(0, 16)] = zeros16
                return carry

            lax.fori_loop(0, sb * 128, zr, 0)

            def stage_body(i, carry):
                off = pl.multiple_of((s + 16 * i) * oc, 8)
                sl = pl.ds(off, oc)

                @pl.when(c == 0)
                def _():
                    pltpu.sync_copy(ys[p][0].at[sl], y_sp.at[sl])

                @pl.when(c == 1)
                def _():
                    pltpu.sync_copy(ys[p][1].at[sl], y_sp.at[sl])

                pltpu.sync_copy(rows_v.at[pl.ds(0, oc)], agg_sp.at[sl])
                return carry

            lax.fori_loop(0, nk, stage_body, 0)
            plsc.subcore_barrier()

            def batch_body(g, carry):
                chunk0 = s * chunks_per_tile + g * sb
                e0 = chunk0 * 128
                pltpu.sync_copy(src_hbm.at[pl.ds(chunk0, sb)], src_v)
                pltpu.sync_copy(dst_hbm.at[pl.ds(chunk0, sb)], dst_v)

                @pl.when(c == 0)
                def _():
                    pltpu.sync_copy(ws[p][0].at[pl.ds(e0, sb * 128)], wf_v)

                @pl.when(c == 1)
                def _():
                    pltpu.sync_copy(ws[p][1].at[pl.ds(e0, sb * 128)], wf_v)

                gathers = [
                    pltpu.async_copy(y_sp.at[src_v.at[j]],
                                     rows_v.at[pl.ds(j * 128, 128)], sem)
                    for j in range(sb)
                ]
                for g_ in gathers:
                    g_.wait()

                def mul_body(i, carry2):
                    for u in range(8):
                        e = i * 8 + u
                        wf_v[e, pl.ds(0, 16)] = (
                            rows_v[e, pl.ds(0, 16)] * wf_v[e, pl.ds(0, 16)])
                    return carry2

                lax.fori_loop(0, sb * 16, mul_body, 0)
                for j in range(sb):
                    pltpu.sync_copy(wf_v.at[pl.ds(j * 128, 128)],
                                    agg_sp.at[dst_v.at[j]], add=True)
                return carry

            lax.fori_loop(0, n_batches, batch_body, 0)
            plsc.subcore_barrier()

            def out_body(i, carry):
                off = pl.multiple_of((s + 16 * i) * oc, 8)
                sl = pl.ds(off, oc)

                @pl.when(c == 0)
                def _():
                    pltpu.sync_copy(agg_sp.at[sl], outs[p][0].at[sl])

                @pl.when(c == 1)
                def _():
                    pltpu.sync_copy(agg_sp.at[sl], outs[p][1].at[sl])
                return carry

            lax.fori_loop(0, nk, out_body, 0)
            plsc.subcore_barrier()

    return body(src_mat, dst_mat, *yq, *wfq)


def _out_body(a0_ref, a1_ref, a2_ref, a3_ref, W0_ref, W1_ref, W2_ref, W3_ref,
              b_ref, out_ref):
    acc = jnp.dot(a0_ref[...], W0_ref[...], preferred_element_type=jnp.float32)
    acc += jnp.dot(a1_ref[...], W1_ref[...], preferred_element_type=jnp.float32)
    acc += jnp.dot(a2_ref[...], W2_ref[...], preferred_element_type=jnp.float32)
    acc += jnp.dot(a3_ref[...], W3_ref[...], preferred_element_type=jnp.float32)
    out_ref[...] = _swish(acc + b_ref[...])


def _compute_out(aggq, Wq, br, blk):
    n = aggq[0].shape[0]
    return pl.pallas_call(
        _out_body,
        grid=(n // blk,),
        in_specs=(
            [pl.BlockSpec((blk, 16), lambda i: (i, 0))] * 4
            + [pl.BlockSpec((16, 64), lambda i: (0, 0))] * 4
            + [pl.BlockSpec(br.shape, lambda i: (0, 0))]
        ),
        out_specs=pl.BlockSpec((blk, 64), lambda i: (i, 0)),
        out_shape=jax.ShapeDtypeStruct((n, 64), jnp.float32),
    )(*aggq, *Wq, br)


def kernel(feat, edge_index, fij, rij, W1, b1, W2, b2, W_in2f, W_f2out, b_f2out):
    n = feat.shape[0]
    e = fij.shape[0]
    e_pad = ((e + 16384 - 1) // 16384) * 16384
    pad = e_pad - e

    src = edge_index[0]
    dst = edge_index[1]
    fij_p = jnp.pad(fij, ((0, pad), (0, 0)))
    rij_p = jnp.pad(rij, (0, pad), constant_values=2.0 * _CUTOFF)
    src_mat = jnp.pad(src, (0, pad)).reshape(-1, 128)
    dst_mat = jnp.pad(dst, (0, pad)).reshape(-1, 128)

    wfq = _compute_wf(fij_p, rij_p[:, None], W1, b1[None, :], W2,
                      b2[None, :], blk=2048)
    yq = _compute_y(feat, W_in2f, blk=1000)

    aggq = _sc_middle(src_mat, dst_mat, yq, wfq)

    Wq = [W_f2out[16 * q:16 * (q + 1)] for q in range(4)]
    out = _compute_out(aggq, Wq, b_f2out[None, :], blk=1000)
    return out


# speedup vs baseline: 1.9290x; 1.0955x over previous
import functools

import jax
import jax.numpy as jnp
from jax import lax
from jax.experimental import pallas as pl
from jax.experimental.pallas import tpu as pltpu
import jax.experimental.pallas.tpu_sc as plsc

_CUTOFF = 5.0


def _swish(x):
    return x * jax.nn.sigmoid(x)


def _wf_body(fij_ref, rij_ref, W1_ref, b1_ref, W2_ref, b2_ref, out_ref):
    h = jnp.dot(fij_ref[...], W1_ref[...], preferred_element_type=jnp.float32)
    h = _swish(h + b1_ref[...])
    wf = jnp.dot(h, W2_ref[...], preferred_element_type=jnp.float32) + b2_ref[...]
    r = rij_ref[...]
    c = jnp.where(r < _CUTOFF, 0.5 * (jnp.cos(r * (jnp.pi / _CUTOFF)) + 1.0), 0.0)
    wf = wf * c
    out_ref[...] = jnp.concatenate([wf, jnp.zeros_like(wf)], axis=1)


def _compute_wf(fij, rij2, W1, b1r, W2, b2r, blk):
    e = fij.shape[0]
    return pl.pallas_call(
        _wf_body,
        grid=(e // blk,),
        in_specs=[
            pl.BlockSpec((blk, fij.shape[1]), lambda i: (i, 0)),
            pl.BlockSpec((blk, 1), lambda i: (i, 0)),
            pl.BlockSpec(W1.shape, lambda i: (0, 0)),
            pl.BlockSpec(b1r.shape, lambda i: (0, 0)),
            pl.BlockSpec(W2.shape, lambda i: (0, 0)),
            pl.BlockSpec(b2r.shape, lambda i: (0, 0)),
        ],
        out_specs=pl.BlockSpec((blk, 128), lambda i: (i, 0)),
        out_shape=jax.ShapeDtypeStruct((e, 128), jnp.float32),
    )(fij, rij2, W1, b1r, W2, b2r)


def _proj_body(x_ref, W_ref, out_ref):
    y = jnp.dot(x_ref[...], W_ref[...], preferred_element_type=jnp.float32)
    out_ref[...] = jnp.concatenate([y, jnp.zeros_like(y)], axis=1)


def _compute_y(feat, W_in2f, blk):
    n = feat.shape[0]
    return pl.pallas_call(
        _proj_body,
        grid=(n // blk,),
        in_specs=[
            pl.BlockSpec((blk, feat.shape[1]), lambda i: (i, 0)),
            pl.BlockSpec(W_in2f.shape, lambda i: (0, 0)),
        ],
        out_specs=pl.BlockSpec((blk, 128), lambda i: (i, 0)),
        out_shape=jax.ShapeDtypeStruct((n, 128), jnp.float32),
    )(feat, W_in2f)


def _sc_middle(src_mat, dst_mat, y128, wf128):
    n = y128.shape[0]
    n_chunks = src_mat.shape[0]
    cpt = (n_chunks + 15) // 16
    oc = 400
    n_oc = n // oc
    assert n % oc == 0 and oc % 8 == 0

    mesh = plsc.VectorSubcoreMesh(core_axis_name="c", subcore_axis_name="s")

    @functools.partial(
        pl.kernel,
        out_type=jax.ShapeDtypeStruct((n, 128), jnp.float32),
        mesh=mesh,
        compiler_params=pltpu.CompilerParams(use_tc_tiling_on_sc=False),
        scratch_types=[
            pltpu.VMEM((128,), jnp.int32),
            pltpu.VMEM((128,), jnp.int32),
            pltpu.VMEM((128, 16), jnp.float32),
            pltpu.VMEM((128, 16), jnp.float32),
            pltpu.VMEM((oc, 16), jnp.float32),
            pltpu.VMEM_SHARED((n, 16), jnp.float32),
            pltpu.VMEM_SHARED((n, 16), jnp.float32),
            pltpu.SemaphoreType.DMA,
        ],
    )
    def body(src_hbm, dst_hbm, y_hbm, wf_hbm, out_hbm,
             src_v, dst_v, wf_v, rows_v, zero_v, y_sp, agg_sp, sem):
        c = lax.axis_index("c")
        s = lax.axis_index("s")
        zeros16 = jnp.zeros((16,), jnp.float32)
        nk = (n_oc - s + 15) // 16

        def zb(i, carry):
            zero_v[i, pl.ds(0, 16)] = zeros16
            return carry

        lax.fori_loop(0, oc, zb, 0)

        for p in range(2):
            def stage_body(i, carry):
                off = pl.multiple_of((s + 16 * i) * oc, 8)
                sl = pl.ds(off, oc)

                @pl.when(c == 0)
                def _():
                    pltpu.sync_copy(y_hbm.at[sl, pl.ds(16 * p, 16)],
                                    y_sp.at[sl])

                @pl.when(c == 1)
                def _():
                    pltpu.sync_copy(y_hbm.at[sl, pl.ds(16 * (2 + p), 16)],
                                    y_sp.at[sl])

                pltpu.sync_copy(zero_v, agg_sp.at[sl])
                return carry

            lax.fori_loop(0, nk, stage_body, 0)
            plsc.subcore_barrier()

            def batch_body(i, carry):
                chunk = s * cpt + i

                @pl.when(chunk < n_chunks)
                def _():
                    pltpu.sync_copy(src_hbm.at[chunk], src_v)
                    pltpu.sync_copy(dst_hbm.at[chunk], dst_v)
                    e0 = pl.multiple_of(chunk * 128, 8)

                    @pl.when(c == 0)
                    def _():
                        pltpu.sync_copy(
                            wf_hbm.at[pl.ds(e0, 128), pl.ds(16 * p, 16)],
                            wf_v)

                    @pl.when(c == 1)
                    def _():
                        pltpu.sync_copy(
                            wf_hbm.at[pl.ds(e0, 128), pl.ds(16 * (2 + p), 16)],
                            wf_v)

                    pltpu.async_copy(y_sp.at[src_v], rows_v, sem).wait()

                    def mul_body(i2, carry2):
                        for u in range(8):
                            e = i2 * 8 + u
                            rows_v[e, pl.ds(0, 16)] = (
                                rows_v[e, pl.ds(0, 16)]
                                * wf_v[e, pl.ds(0, 16)])
                        return carry2

                    lax.fori_loop(0, 16, mul_body, 0)
                    pltpu.sync_copy(rows_v, agg_sp.at[dst_v], add=True)
                return carry

            lax.fori_loop(0, cpt, batch_body, 0)
            plsc.subcore_barrier()

            def out_body(i, carry):
                off = pl.multiple_of((s + 16 * i) * oc, 8)
                sl = pl.ds(off, oc)

                @pl.when(c == 0)
                def _():
                    pltpu.sync_copy(agg_sp.at[sl],
                                    out_hbm.at[sl, pl.ds(16 * p, 16)])

                @pl.when(c == 1)
                def _():
                    pltpu.sync_copy(agg_sp.at[sl],
                                    out_hbm.at[sl, pl.ds(16 * (2 + p), 16)])
                return carry

            lax.fori_loop(0, nk, out_body, 0)
            plsc.subcore_barrier()

    return body(src_mat, dst_mat, y128, wf128)


def _out_body(agg_ref, W_ref, b_ref, out_ref):
    acc = jnp.dot(agg_ref[...][:, :64], W_ref[...],
                  preferred_element_type=jnp.float32)
    out_ref[...] = _swish(acc + b_ref[...])


def _compute_out(agg128, W_f2out, br, blk):
    n = agg128.shape[0]
    return pl.pallas_call(
        _out_body,
        grid=(n // blk,),
        in_specs=[
            pl.BlockSpec((blk, 128), lambda i: (i, 0)),
            pl.BlockSpec(W_f2out.shape, lambda i: (0, 0)),
            pl.BlockSpec(br.shape, lambda i: (0, 0)),
        ],
        out_specs=pl.BlockSpec((blk, 64), lambda i: (i, 0)),
        out_shape=jax.ShapeDtypeStruct((n, 64), jnp.float32),
    )(agg128, W_f2out, br)


def kernel(feat, edge_index, fij, rij, W1, b1, W2, b2, W_in2f, W_f2out, b_f2out):
    src_mat = edge_index[0].reshape(-1, 128)
    dst_mat = edge_index[1].reshape(-1, 128)

    wf128 = _compute_wf(fij, rij[:, None], W1, b1[None, :], W2,
                        b2[None, :], blk=1600)
    y128 = _compute_y(feat, W_in2f, blk=1000)

    agg128 = _sc_middle(src_mat, dst_mat, y128, wf128)

    out = _compute_out(agg128, W_f2out, b_f2out[None, :], blk=1000)
    return out


# speedup vs baseline: 2.6947x; 1.3970x over previous
import functools

import jax
import jax.numpy as jnp
from jax import lax
from jax.experimental import pallas as pl
from jax.experimental.pallas import tpu as pltpu
import jax.experimental.pallas.tpu_sc as plsc

_CUTOFF = 5.0


def _swish(x):
    return x * jax.nn.sigmoid(x)


def _wf_body(fij_ref, rij_ref, W1_ref, b1_ref, W2_ref, b2_ref, out_ref):
    h = jnp.dot(fij_ref[...], W1_ref[...], preferred_element_type=jnp.float32)
    h = _swish(h + b1_ref[...])
    wf = jnp.dot(h, W2_ref[...], preferred_element_type=jnp.float32) + b2_ref[...]
    r = rij_ref[...]
    c = jnp.where(r < _CUTOFF, 0.5 * (jnp.cos(r * (jnp.pi / _CUTOFF)) + 1.0), 0.0)
    wf = wf * c
    out_ref[...] = jnp.concatenate([wf, jnp.zeros_like(wf)], axis=1)


def _compute_wf(fij, rij2, W1, b1r, W2, b2r, blk):
    e = fij.shape[0]
    return pl.pallas_call(
        _wf_body,
        grid=(e // blk,),
        in_specs=[
            pl.BlockSpec((blk, fij.shape[1]), lambda i: (i, 0)),
            pl.BlockSpec((blk, 1), lambda i: (i, 0)),
            pl.BlockSpec(W1.shape, lambda i: (0, 0)),
            pl.BlockSpec(b1r.shape, lambda i: (0, 0)),
            pl.BlockSpec(W2.shape, lambda i: (0, 0)),
            pl.BlockSpec(b2r.shape, lambda i: (0, 0)),
        ],
        out_specs=pl.BlockSpec((blk, 128), lambda i: (i, 0)),
        out_shape=jax.ShapeDtypeStruct((e, 128), jnp.float32),
    )(fij, rij2, W1, b1r, W2, b2r)


def _proj_body(x_ref, W_ref, out0_ref, out1_ref):
    y = jnp.dot(x_ref[...], W_ref[...], preferred_element_type=jnp.float32)
    out0_ref[...] = y[:, :32]
    out1_ref[...] = y[:, 32:]


def _compute_y(feat, W_in2f, blk):
    n = feat.shape[0]
    half = jax.ShapeDtypeStruct((n, 32), jnp.float32)
    return pl.pallas_call(
        _proj_body,
        grid=(n // blk,),
        in_specs=[
            pl.BlockSpec((blk, feat.shape[1]), lambda i: (i, 0)),
            pl.BlockSpec(W_in2f.shape, lambda i: (0, 0)),
        ],
        out_specs=[pl.BlockSpec((blk, 32), lambda i: (i, 0))] * 2,
        out_shape=[half, half],
    )(feat, W_in2f)


def _sc_middle(src_mat, dst_mat, y0, y1, wf128, n_edges):
    n = y0.shape[0]
    n_chunks = n_edges // 128
    sb = 3
    cpt = src_mat.shape[0] // 16
    n_batches = cpt // sb
    assert n_batches * sb == cpt
    oc = 200
    n_oc = n // oc
    assert n % oc == 0 and oc % 8 == 0

    mesh = plsc.VectorSubcoreMesh(core_axis_name="c", subcore_axis_name="s")

    @functools.partial(
        pl.kernel,
        out_type=jax.ShapeDtypeStruct((n, 128), jnp.float32),
        mesh=mesh,
        compiler_params=pltpu.CompilerParams(use_tc_tiling_on_sc=False),
        scratch_types=[
            pltpu.VMEM((sb, 128), jnp.int32),
            pltpu.VMEM((sb, 128), jnp.int32),
            pltpu.VMEM((sb * 128, 32), jnp.float32),
            pltpu.VMEM((sb * 128, 32), jnp.float32),
            pltpu.VMEM_SHARED((n, 32), jnp.float32),
            pltpu.SemaphoreType.DMA,
            pltpu.SemaphoreType.DMA,
        ],
    )
    def body(src_hbm, dst_hbm, y0_hbm, y1_hbm, wf_hbm, out_hbm,
             src_v, dst_v, wf_v, rows_v, agg_sp, semg, semw):
        c = lax.axis_index("c")
        s = lax.axis_index("s")
        zeros16 = jnp.zeros((16,), jnp.float32)
        nk = (n_oc - s + 15) // 16

        def zb(i, carry):
            rows_v[i, pl.ds(0, 16)] = zeros16
            rows_v[i, pl.ds(16, 16)] = zeros16
            return carry

        lax.fori_loop(0, oc, zb, 0)

        def stage_body(i, carry):
            off = pl.multiple_of((s + 16 * i) * oc, 8)
            pltpu.sync_copy(rows_v.at[pl.ds(0, oc)], agg_sp.at[pl.ds(off, oc)])
            return carry

        lax.fori_loop(0, nk, stage_body, 0)
        plsc.subcore_barrier()

        def batch_body(g, carry):
            chunk0 = s * cpt + g * sb
            pltpu.sync_copy(src_hbm.at[pl.ds(chunk0, sb)], src_v)
            pltpu.sync_copy(dst_hbm.at[pl.ds(chunk0, sb)], dst_v)
            e0 = pl.multiple_of(chunk0 * 128, 8)
            nvalid = jnp.clip(n_chunks - chunk0, 0, sb)

            @pl.when(nvalid == sb)
            def _():
                @pl.when(c == 0)
                def _():
                    pltpu.async_copy(
                        wf_hbm.at[pl.ds(e0, sb * 128), pl.ds(0, 32)],
                        wf_v, semw)

                @pl.when(c == 1)
                def _():
                    pltpu.async_copy(
                        wf_hbm.at[pl.ds(e0, sb * 128), pl.ds(32, 32)],
                        wf_v, semw)

            for j in range(sb):
                @pl.when(j < nvalid)
                def _():
                    rsl = pl.ds(j * 128, 128)

                    @pl.when(nvalid < sb)
                    def _():
                        ej = pl.multiple_of((chunk0 + j) * 128, 8)

                        @pl.when(c == 0)
                        def _():
                            pltpu.async_copy(
                                wf_hbm.at[pl.ds(ej, 128), pl.ds(0, 32)],
                                wf_v.at[rsl], semw)

                        @pl.when(c == 1)
                        def _():
                            pltpu.async_copy(
                                wf_hbm.at[pl.ds(ej, 128), pl.ds(32, 32)],
                                wf_v.at[rsl], semw)

                    @pl.when(c == 0)
                    def _():
                        pltpu.async_copy(y0_hbm.at[src_v.at[j]],
                                         rows_v.at[rsl], semg)

                    @pl.when(c == 1)
                    def _():
                        pltpu.async_copy(y1_hbm.at[src_v.at[j]],
                                         rows_v.at[rsl], semg)

            for j in range(sb):
                @pl.when(j < nvalid)
                def _():
                    rsl = pl.ds(j * 128, 128)
                    pltpu.make_async_copy(y0_hbm.at[src_v.at[j]],
                                          rows_v.at[rsl], semg).wait()

                    @pl.when(nvalid < sb)
                    def _():
                        ej = pl.multiple_of((chunk0 + j) * 128, 8)
                        pltpu.make_async_copy(
                            wf_hbm.at[pl.ds(ej, 128), pl.ds(0, 32)],
                            wf_v.at[rsl], semw).wait()

            @pl.when(nvalid == sb)
            def _():
                pltpu.make_async_copy(
                    wf_hbm.at[pl.ds(e0, sb * 128), pl.ds(0, 32)],
                    wf_v, semw).wait()

            def mul_body(i2, carry2):
                for u in range(4):
                    e = i2 * 4 + u
                    for k in range(2):
                        sl = pl.ds(k * 16, 16)
                        rows_v[e, sl] = rows_v[e, sl] * wf_v[e, sl]
                return carry2

            lax.fori_loop(0, sb * 32, mul_body, 0)

            for j in range(sb):
                @pl.when(j < nvalid)
                def _():
                    pltpu.sync_copy(rows_v.at[pl.ds(j * 128, 128)],
                                    agg_sp.at[dst_v.at[j]], add=True)
            return carry

        lax.fori_loop(0, n_batches, batch_body, 0)
        plsc.subcore_barrier()

        def out_body(i, carry):
            off = pl.multiple_of((s + 16 * i) * oc, 8)
            sl = pl.ds(off, oc)

            @pl.when(c == 0)
            def _():
                pltpu.sync_copy(agg_sp.at[sl], out_hbm.at[sl, pl.ds(0, 32)])

            @pl.when(c == 1)
            def _():
                pltpu.sync_copy(agg_sp.at[sl], out_hbm.at[sl, pl.ds(32, 32)])
            return carry

        lax.fori_loop(0, nk, out_body, 0)

    return body(src_mat, dst_mat, y0, y1, wf128)


def _out_body(agg_ref, W_ref, b_ref, out_ref):
    acc = jnp.dot(agg_ref[...][:, :64], W_ref[...],
                  preferred_element_type=jnp.float32)
    out_ref[...] = _swish(acc + b_ref[...])


def _compute_out(agg128, W_f2out, br, blk):
    n = agg128.shape[0]
    return pl.pallas_call(
        _out_body,
        grid=(n // blk,),
        in_specs=[
            pl.BlockSpec((blk, 128), lambda i: (i, 0)),
            pl.BlockSpec(W_f2out.shape, lambda i: (0, 0)),
            pl.BlockSpec(br.shape, lambda i: (0, 0)),
        ],
        out_specs=pl.BlockSpec((blk, 64), lambda i: (i, 0)),
        out_shape=jax.ShapeDtypeStruct((n, 64), jnp.float32),
    )(agg128, W_f2out, br)


def kernel(feat, edge_index, fij, rij, W1, b1, W2, b2, W_in2f, W_f2out, b_f2out):
    e = fij.shape[0]
    grp = 16 * 3 * 128
    e_virt = ((e + grp - 1) // grp) * grp
    pad = e_virt - e

    src_mat = jnp.pad(edge_index[0], (0, pad)).reshape(-1, 128)
    dst_mat = jnp.pad(edge_index[1], (0, pad)).reshape(-1, 128)

    wf128 = _compute_wf(fij, rij[:, None], W1, b1[None, :], W2,
                        b2[None, :], blk=1600)
    y0, y1 = _compute_y(feat, W_in2f, blk=1000)

    agg128 = _sc_middle(src_mat, dst_mat, y0, y1, wf128, e)

    out = _compute_out(agg128, W_f2out, b_f2out[None, :], blk=1000)
    return out


# speedup vs baseline: 7.4211x; 2.7539x over previous
import functools

import jax
import jax.numpy as jnp
from jax import lax
from jax.experimental import pallas as pl
from jax.experimental.pallas import tpu as pltpu
import jax.experimental.pallas.tpu_sc as plsc

_CUTOFF = 5.0


def _swish(x):
    return x * jax.nn.sigmoid(x)


def _wf_body(fijT_ref, rij_ref, W1_ref, b1_ref, W2_ref, b2_ref, out_ref):
    h = lax.dot_general(fijT_ref[...], W1_ref[...], (((0,), (0,)), ((), ())),
                        preferred_element_type=jnp.float32)
    h = _swish(h + b1_ref[...])
    wf = jnp.dot(h, W2_ref[...], preferred_element_type=jnp.float32) + b2_ref[...]
    r = rij_ref[...]
    c = jnp.where(r < _CUTOFF, 0.5 * (jnp.cos(r * (jnp.pi / _CUTOFF)) + 1.0), 0.0)
    wf = wf * c[:, None]
    out_ref[...] = jnp.concatenate([wf, jnp.zeros_like(wf)], axis=1)


def _compute_wf(fijT, rij, W1, b1r, W2, b2r, blk):
    e = fijT.shape[1]
    return pl.pallas_call(
        _wf_body,
        grid=(pl.cdiv(e, blk),),
        in_specs=[
            pl.BlockSpec((fijT.shape[0], blk), lambda i: (0, i)),
            pl.BlockSpec((blk,), lambda i: (i,)),
            pl.BlockSpec(W1.shape, lambda i: (0, 0)),
            pl.BlockSpec(b1r.shape, lambda i: (0, 0)),
            pl.BlockSpec(W2.shape, lambda i: (0, 0)),
            pl.BlockSpec(b2r.shape, lambda i: (0, 0)),
        ],
        out_specs=pl.BlockSpec((blk, 128), lambda i: (i, 0)),
        out_shape=jax.ShapeDtypeStruct((e, 128), jnp.float32),
    )(fijT, rij, W1, b1r, W2, b2r)


def _proj_body(xT_ref, W_ref, out0_ref, out1_ref):
    y = lax.dot_general(xT_ref[...], W_ref[...], (((0,), (0,)), ((), ())),
                        preferred_element_type=jnp.float32)
    out0_ref[...] = y[:, :32]
    out1_ref[...] = y[:, 32:]


def _compute_y(featT, W_in2f, blk):
    n = featT.shape[1]
    half = jax.ShapeDtypeStruct((n, 32), jnp.float32)
    return pl.pallas_call(
        _proj_body,
        grid=(pl.cdiv(n, blk),),
        in_specs=[
            pl.BlockSpec((featT.shape[0], blk), lambda i: (0, i)),
            pl.BlockSpec(W_in2f.shape, lambda i: (0, 0)),
        ],
        out_specs=[pl.BlockSpec((blk, 32), lambda i: (i, 0))] * 2,
        out_shape=[half, half],
    )(featT, W_in2f)


def _sc_middle(src_mat, dst_mat, y0, y1, wf128, n_edges):
    n = y0.shape[0]
    n_chunks = n_edges // 128
    sb = 3
    cpt = src_mat.shape[0] // 16
    n_batches = cpt // sb
    assert n_batches * sb == cpt
    oc = 200
    n_oc = n // oc
    assert n % oc == 0 and oc % 8 == 0

    mesh = plsc.VectorSubcoreMesh(core_axis_name="c", subcore_axis_name="s")

    @functools.partial(
        pl.kernel,
        out_type=jax.ShapeDtypeStruct((n, 128), jnp.float32),
        mesh=mesh,
        compiler_params=pltpu.CompilerParams(use_tc_tiling_on_sc=False),
        scratch_types=[
            pltpu.VMEM((sb, 128), jnp.int32),
            pltpu.VMEM((sb, 128), jnp.int32),
            pltpu.VMEM((sb * 128, 32), jnp.float32),
            pltpu.VMEM((sb * 128, 32), jnp.float32),
            pltpu.VMEM_SHARED((n, 32), jnp.float32),
            pltpu.SemaphoreType.DMA,
            pltpu.SemaphoreType.DMA,
        ],
    )
    def body(src_hbm, dst_hbm, y0_hbm, y1_hbm, wf_hbm, out_hbm,
             src_v, dst_v, wf_v, rows_v, agg_sp, semg, semw):
        c = lax.axis_index("c")
        s = lax.axis_index("s")
        zeros16 = jnp.zeros((16,), jnp.float32)
        nk = (n_oc - s + 15) // 16

        def zb(i, carry):
            rows_v[i, pl.ds(0, 16)] = zeros16
            rows_v[i, pl.ds(16, 16)] = zeros16
            return carry

        lax.fori_loop(0, oc, zb, 0)

        def stage_body(i, carry):
            off = pl.multiple_of((s + 16 * i) * oc, 8)
            pltpu.sync_copy(rows_v.at[pl.ds(0, oc)], agg_sp.at[pl.ds(off, oc)])
            return carry

        lax.fori_loop(0, nk, stage_body, 0)
        plsc.subcore_barrier()

        def batch_body(g, carry):
            chunk0 = s * cpt + g * sb
            pltpu.sync_copy(src_hbm.at[pl.ds(chunk0, sb)], src_v)
            pltpu.sync_copy(dst_hbm.at[pl.ds(chunk0, sb)], dst_v)
            e0 = pl.multiple_of(chunk0 * 128, 8)
            nvalid = jnp.clip(n_chunks - chunk0, 0, sb)

            @pl.when(nvalid == sb)
            def _():
                @pl.when(c == 0)
                def _():
                    pltpu.async_copy(
                        wf_hbm.at[pl.ds(e0, sb * 128), pl.ds(0, 32)],
                        wf_v, semw)

                @pl.when(c == 1)
                def _():
                    pltpu.async_copy(
                        wf_hbm.at[pl.ds(e0, sb * 128), pl.ds(32, 32)],
                        wf_v, semw)

            for j in range(sb):
                @pl.when(j < nvalid)
                def _():
                    rsl = pl.ds(j * 128, 128)

                    @pl.when(nvalid < sb)
                    def _():
                        ej = pl.multiple_of((chunk0 + j) * 128, 8)

                        @pl.when(c == 0)
                        def _():
                            pltpu.async_copy(
                                wf_hbm.at[pl.ds(ej, 128), pl.ds(0, 32)],
                                wf_v.at[rsl], semw)

                        @pl.when(c == 1)
                        def _():
                            pltpu.async_copy(
                                wf_hbm.at[pl.ds(ej, 128), pl.ds(32, 32)],
                                wf_v.at[rsl], semw)

                    @pl.when(c == 0)
                    def _():
                        pltpu.async_copy(y0_hbm.at[src_v.at[j]],
                                         rows_v.at[rsl], semg)

                    @pl.when(c == 1)
                    def _():
                        pltpu.async_copy(y1_hbm.at[src_v.at[j]],
                                         rows_v.at[rsl], semg)

            for j in range(sb):
                @pl.when(j < nvalid)
                def _():
                    rsl = pl.ds(j * 128, 128)
                    pltpu.make_async_copy(y0_hbm.at[src_v.at[j]],
                                          rows_v.at[rsl], semg).wait()

                    @pl.when(nvalid < sb)
                    def _():
                        ej = pl.multiple_of((chunk0 + j) * 128, 8)
                        pltpu.make_async_copy(
                            wf_hbm.at[pl.ds(ej, 128), pl.ds(0, 32)],
                            wf_v.at[rsl], semw).wait()

            @pl.when(nvalid == sb)
            def _():
                pltpu.make_async_copy(
                    wf_hbm.at[pl.ds(e0, sb * 128), pl.ds(0, 32)],
                    wf_v, semw).wait()

            def mul_body(i2, carry2):
                for u in range(4):
                    e = i2 * 4 + u
                    for k in range(2):
                        sl = pl.ds(k * 16, 16)
                        rows_v[e, sl] = rows_v[e, sl] * wf_v[e, sl]
                return carry2

            lax.fori_loop(0, sb * 32, mul_body, 0)

            for j in range(sb):
                @pl.when(j < nvalid)
                def _():
                    pltpu.sync_copy(rows_v.at[pl.ds(j * 128, 128)],
                                    agg_sp.at[dst_v.at[j]], add=True)
            return carry

        lax.fori_loop(0, n_batches, batch_body, 0)
        plsc.subcore_barrier()

        def out_body(i, carry):
            off = pl.multiple_of((s + 16 * i) * oc, 8)
            sl = pl.ds(off, oc)

            @pl.when(c == 0)
            def _():
                pltpu.sync_copy(agg_sp.at[sl], out_hbm.at[sl, pl.ds(0, 32)])

            @pl.when(c == 1)
            def _():
                pltpu.sync_copy(agg_sp.at[sl], out_hbm.at[sl, pl.ds(32, 32)])
            return carry

        lax.fori_loop(0, nk, out_body, 0)

    return body(src_mat, dst_mat, y0, y1, wf128)


def _out_body(agg_ref, W_ref, b_ref, out_ref):
    acc = jnp.dot(agg_ref[...][:, :64], W_ref[...],
                  preferred_element_type=jnp.float32)
    out_ref[...] = _swish(acc + b_ref[...])


def _compute_out(agg128, W_f2out, br, blk):
    n = agg128.shape[0]
    return pl.pallas_call(
        _out_body,
        grid=(n // blk,),
        in_specs=[
            pl.BlockSpec((blk, 128), lambda i: (i, 0)),
            pl.BlockSpec(W_f2out.shape, lambda i: (0, 0)),
            pl.BlockSpec(br.shape, lambda i: (0, 0)),
        ],
        out_specs=pl.BlockSpec((blk, 64), lambda i: (i, 0)),
        out_shape=jax.ShapeDtypeStruct((n, 64), jnp.float32),
    )(agg128, W_f2out, br)


def kernel(feat, edge_index, fij, rij, W1, b1, W2, b2, W_in2f, W_f2out, b_f2out):
    e = fij.shape[0]
    grp = 16 * 3 * 128
    e_virt = ((e + grp - 1) // grp) * grp
    pad = e_virt - e

    src_mat = jnp.pad(edge_index[0], (0, pad)).reshape(-1, 128)
    dst_mat = jnp.pad(edge_index[1], (0, pad)).reshape(-1, 128)

    wf128 = _compute_wf(fij.T, rij, W1, b1[None, :], W2,
                        b2[None, :], blk=4096)
    y0, y1 = _compute_y(feat.T, W_in2f, blk=3200)

    agg128 = _sc_middle(src_mat, dst_mat, y0, y1, wf128, e)

    out = _compute_out(agg128, W_f2out, b_f2out[None, :], blk=1000)
    return out


# speedup vs baseline: 7.4770x; 1.0075x over previous
import functools

import jax
import jax.numpy as jnp
from jax import lax
from jax.experimental import pallas as pl
from jax.experimental.pallas import tpu as pltpu
import jax.experimental.pallas.tpu_sc as plsc

_CUTOFF = 5.0


def _swish(x):
    return x * jax.nn.sigmoid(x)


def _wf_body(fijT_ref, rij_ref, W1_ref, b1_ref, W2_ref, b2_ref, out_ref):
    h = lax.dot_general(fijT_ref[...], W1_ref[...], (((0,), (0,)), ((), ())),
                        preferred_element_type=jnp.float32)
    h = _swish(h + b1_ref[...])
    wf = jnp.dot(h, W2_ref[...], preferred_element_type=jnp.float32) + b2_ref[...]
    r = rij_ref[...]
    c = jnp.where(r < _CUTOFF, 0.5 * (jnp.cos(r * (jnp.pi / _CUTOFF)) + 1.0), 0.0)
    wf = wf * c[:, None]
    out_ref[...] = jnp.concatenate([wf, jnp.zeros_like(wf)], axis=1)


def _compute_wf(fijT, rij, W1, b1r, W2, b2r, blk):
    e = fijT.shape[1]
    return pl.pallas_call(
        _wf_body,
        grid=(pl.cdiv(e, blk),),
        in_specs=[
            pl.BlockSpec((fijT.shape[0], blk), lambda i: (0, i)),
            pl.BlockSpec((blk,), lambda i: (i,)),
            pl.BlockSpec(W1.shape, lambda i: (0, 0)),
            pl.BlockSpec(b1r.shape, lambda i: (0, 0)),
            pl.BlockSpec(W2.shape, lambda i: (0, 0)),
            pl.BlockSpec(b2r.shape, lambda i: (0, 0)),
        ],
        out_specs=pl.BlockSpec((blk, 128), lambda i: (i, 0)),
        out_shape=jax.ShapeDtypeStruct((e, 128), jnp.float32),
    )(fijT, rij, W1, b1r, W2, b2r)


def _proj_body(xT_ref, W_ref, out0_ref, out1_ref):
    y = lax.dot_general(xT_ref[...], W_ref[...], (((0,), (0,)), ((), ())),
                        preferred_element_type=jnp.float32)
    out0_ref[...] = y[:, :32]
    out1_ref[...] = y[:, 32:]


def _compute_y(featT, W_in2f, blk):
    n = featT.shape[1]
    half = jax.ShapeDtypeStruct((n, 32), jnp.float32)
    return pl.pallas_call(
        _proj_body,
        grid=(pl.cdiv(n, blk),),
        in_specs=[
            pl.BlockSpec((featT.shape[0], blk), lambda i: (0, i)),
            pl.BlockSpec(W_in2f.shape, lambda i: (0, 0)),
        ],
        out_specs=[pl.BlockSpec((blk, 32), lambda i: (i, 0))] * 2,
        out_shape=[half, half],
    )(featT, W_in2f)


def _sc_middle(src_mat, dst_mat, y0, y1, wf128, n_edges):
    n = y0.shape[0]
    n_chunks = n_edges // 128
    cpt = src_mat.shape[0] // 16
    assert cpt % 2 == 0
    oc = 80
    n_oc = n // oc
    oco = 400
    n_oco = n // oco
    assert n % oc == 0 and oc % 8 == 0 and n % oco == 0 and oco % 8 == 0

    mesh = plsc.VectorSubcoreMesh(core_axis_name="c", subcore_axis_name="s")

    @functools.partial(
        pl.kernel,
        out_type=jax.ShapeDtypeStruct((n, 128), jnp.float32),
        mesh=mesh,
        compiler_params=pltpu.CompilerParams(use_tc_tiling_on_sc=False),
        scratch_types=[
            pltpu.VMEM((128,), jnp.int32),
            pltpu.VMEM((128,), jnp.int32),
            pltpu.VMEM((128, 32), jnp.float32),
            pltpu.VMEM((128, 32), jnp.float32),
            pltpu.VMEM((128,), jnp.int32),
            pltpu.VMEM((128,), jnp.int32),
            pltpu.VMEM((128, 32), jnp.float32),
            pltpu.VMEM((128, 32), jnp.float32),
            pltpu.VMEM_SHARED((n, 32), jnp.float32),
            pltpu.SemaphoreType.DMA,
            pltpu.SemaphoreType.DMA,
            pltpu.SemaphoreType.DMA,
            pltpu.SemaphoreType.DMA,
            pltpu.SemaphoreType.DMA,
            pltpu.SemaphoreType.DMA,
        ],
    )
    def body(src_hbm, dst_hbm, y0_hbm, y1_hbm, wf_hbm, out_hbm,
             srcA, dstA, wfA, rowsA, srcB, dstB, wfB, rowsB, agg_sp,
             semIA, semWA, semGA, semIB, semWB, semGB):
        c = lax.axis_index("c")
        s = lax.axis_index("s")
        base = s * cpt
        zeros16 = jnp.zeros((16,), jnp.float32)
        nk = (n_oc - s + 15) // 16

        def zb(i, carry):
            rowsA[i, pl.ds(0, 16)] = zeros16
            rowsA[i, pl.ds(16, 16)] = zeros16
            return carry

        lax.fori_loop(0, oc, zb, 0)

        def stage_body(i, carry):
            off = pl.multiple_of((s + 16 * i) * oc, 8)
            pltpu.sync_copy(rowsA.at[pl.ds(0, oc)], agg_sp.at[pl.ds(off, oc)])
            return carry

        lax.fori_loop(0, nk, stage_body, 0)
        plsc.subcore_barrier()

        def ok_for(i):
            return jnp.logical_and(i < cpt, base + i < n_chunks)

        def fire_front(i, srcb, dstb, wfb, semI, semW):
            chunk = base + i

            @pl.when(ok_for(i))
            def _():
                pltpu.async_copy(src_hbm.at[chunk], srcb, semI)
                pltpu.async_copy(dst_hbm.at[chunk], dstb, semI)
                e0 = pl.multiple_of(chunk * 128, 8)

                @pl.when(c == 0)
                def _():
                    pltpu.async_copy(
                        wf_hbm.at[pl.ds(e0, 128), pl.ds(0, 32)], wfb, semW)

                @pl.when(c == 1)
                def _():
                    pltpu.async_copy(
                        wf_hbm.at[pl.ds(e0, 128), pl.ds(32, 32)], wfb, semW)

        def fire_gather(i, srcb, dstb, rowsb, semI, semG):
            chunk = base + i

            @pl.when(ok_for(i))
            def _():
                pltpu.make_async_copy(src_hbm.at[chunk], srcb, semI).wait()
                pltpu.make_async_copy(dst_hbm.at[chunk], dstb, semI).wait()

                @pl.when(c == 0)
                def _():
                    pltpu.async_copy(y0_hbm.at[srcb], rowsb, semG)

                @pl.when(c == 1)
                def _():
                    pltpu.async_copy(y1_hbm.at[srcb], rowsb, semG)

        def process(i, srcb, dstb, wfb, rowsb, semW, semG):
            chunk = base + i

            @pl.when(ok_for(i))
            def _():
                pltpu.make_async_copy(y0_hbm.at[srcb], rowsb, semG).wait()
                e0 = pl.multiple_of(chunk * 128, 8)
                pltpu.make_async_copy(
                    wf_hbm.at[pl.ds(e0, 128), pl.ds(0, 32)], wfb, semW).wait()

                def mul_body(i2, carry2):
                    for u in range(8):
                        e = i2 * 8 + u
                        for k in range(2):
                            sl = pl.ds(k * 16, 16)
                            rowsb[e, sl] = rowsb[e, sl] * wfb[e, sl]
                    return carry2

                lax.fori_loop(0, 16, mul_body, 0)
                pltpu.sync_copy(rowsb, agg_sp.at[dstb], add=True)

        A = (srcA, dstA, wfA, rowsA, semIA, semWA, semGA)
        B = (srcB, dstB, wfB, rowsB, semIB, semWB, semGB)

        def ff(i, t):
            fire_front(i, t[0], t[1], t[2], t[4], t[5])

        def fg(i, t):
            fire_gather(i, t[0], t[1], t[3], t[4], t[6])

        def pr(i, t):
            process(i, t[0], t[1], t[2], t[3], t[5], t[6])

        ff(0, A)
        ff(1, B)
        fg(0, A)

        def pair_body(j, carry):
            i0 = 2 * j
            pr(i0, A)
            ff(i0 + 2, A)
            fg(i0 + 1, B)
            pr(i0 + 1, B)
            ff(i0 + 3, B)
            fg(i0 + 2, A)
            return carry

        lax.fori_loop(0, cpt // 2, pair_body, 0)
        plsc.subcore_barrier()

        nko = (n_oco - s + 15) // 16

        def out_body(i, carry):
            off = pl.multiple_of((s + 16 * i) * oco, 8)
            sl = pl.ds(off, oco)

            @pl.when(c == 0)
            def _():
                pltpu.sync_copy(agg_sp.at[sl], out_hbm.at[sl, pl.ds(0, 32)])

            @pl.when(c == 1)
            def _():
                pltpu.sync_copy(agg_sp.at[sl], out_hbm.at[sl, pl.ds(32, 32)])
            return carry

        lax.fori_loop(0, nko, out_body, 0)

    return body(src_mat, dst_mat, y0, y1, wf128)


def _out_body(agg_ref, W_ref, b_ref, out_ref):
    acc = jnp.dot(agg_ref[...][:, :64], W_ref[...],
                  preferred_element_type=jnp.float32)
    out_ref[...] = _swish(acc + b_ref[...])


def _compute_out(agg128, W_f2out, br, blk):
    n = agg128.shape[0]
    return pl.pallas_call(
        _out_body,
        grid=(n // blk,),
        in_specs=[
            pl.BlockSpec((blk, 128), lambda i: (i, 0)),
            pl.BlockSpec(W_f2out.shape, lambda i: (0, 0)),
            pl.BlockSpec(br.shape, lambda i: (0, 0)),
        ],
        out_specs=pl.BlockSpec((blk, 64), lambda i: (i, 0)),
        out_shape=jax.ShapeDtypeStruct((n, 64), jnp.float32),
    )(agg128, W_f2out, br)


def kernel(feat, edge_index, fij, rij, W1, b1, W2, b2, W_in2f, W_f2out, b_f2out):
    e = fij.shape[0]
    grp = 16 * 2 * 128
    e_virt = ((e + grp - 1) // grp) * grp
    pad = e_virt - e

    src_mat = jnp.pad(edge_index[0], (0, pad)).reshape(-1, 128)
    dst_mat = jnp.pad(edge_index[1], (0, pad)).reshape(-1, 128)

    wf128 = _compute_wf(fij.T, rij, W1, b1[None, :], W2,
                        b2[None, :], blk=4096)
    y0, y1 = _compute_y(feat.T, W_in2f, blk=3200)

    agg128 = _sc_middle(src_mat, dst_mat, y0, y1, wf128, e)

    out = _compute_out(agg128, W_f2out, b_f2out[None, :], blk=1000)
    return out
